# Initial kernel scaffold; baseline (speedup 1.0000x reference)
import jax
import jax.numpy as jnp
from jax.experimental import pallas as pl


def kernel(x0, x1, x2, idf_logits):
    raise NotImplementedError("write your pallas kernel here")


# speedup vs baseline: 1.7230x; 1.7230x over previous
import jax
import jax.numpy as jnp
import numpy as np
from jax.experimental import pallas as pl
from jax.experimental.pallas import tpu as pltpu

_ANCH = (
    ((10.0, 13.0), (16.0, 30.0), (33.0, 23.0)),
    ((30.0, 61.0), (62.0, 45.0), (59.0, 119.0)),
    ((116.0, 90.0), (156.0, 198.0), (373.0, 326.0)),
)
_W = (76, 38, 19)
_HW = (5776, 1444, 361)
_HWP = (5888, 1536, 384)
_NCH = (46, 12, 3)
_J0 = (0, 46, 58)
_ROW0 = (0, 17328, 21660)
_STRIDE = (8.0, 16.0, 32.0)
_ROWS = 22743


def _build_add_table():
    blocks = []
    for s in range(3):
        w, st = _W[s], _STRIDE[s]
        hw = np.arange(_NCH[s] * 128)
        t = np.zeros((_NCH[s] * 128, 85), np.float32)
        t[:, 0] = (hw % w) * st
        t[:, 1] = (hw // w) * st
        blocks.append(t.reshape(_NCH[s], 128, 85))
    return np.concatenate(blocks, 0)


_ADD_TAB = _build_add_table()


def _mul_const(s, a):
    m = np.ones((1, 85), np.float32)
    m[0, 0] = m[0, 1] = _STRIDE[s]
    m[0, 2] = _ANCH[s][a][0]
    m[0, 3] = _ANCH[s][a][1]
    return m


_MUL_TAB = np.concatenate(
    [_mul_const(s, a) for s in range(3) for a in range(3)], 0
).reshape(9, 1, 85)


def _body(x0_ref, x1_ref, x2_ref, add_ref, pre_ref, mul_ref, out_ref, scr_ref):
    j = pl.program_id(1)
    lane = jax.lax.broadcasted_iota(jnp.int32, (128, 85), 1)
    isexp = (lane == 2) | (lane == 3)
    pre = pre_ref[...]
    add = add_ref[0]

    for s, x_ref in enumerate((x0_ref, x1_ref, x2_ref)):
        jlo, nch = _J0[s], _NCH[s]

        @pl.when((j >= jlo) & (j < jlo + nch))
        def _(s=s, x_ref=x_ref, jlo=jlo, nch=nch):
            t_all = jnp.transpose(x_ref[0])
            for a in range(3):
                u = t_all[:, a * 85:(a + 1) * 85] * pre
                e = jnp.exp(u)
                nl = jnp.where(isexp, e, 1.0 / (1.0 + e))
                o = nl * mul_ref[s * 3 + a] + add
                scr_ref[a::3, :] = o
            k = j - jlo
            base = _ROW0[s]
            tail_rows = (_HW[s] - (nch - 1) * 128) * 3
            if s == 2:
                for kk in range(nch):
                    @pl.when(k == kk)
                    def _(kk=kk):
                        r0 = base + kk * 384
                        r1 = min(base + _HW[s] * 3, r0 + 384)
                        out_ref[0, r0:r1, :] = scr_ref[:r1 - r0, :]
            else:
                @pl.when(k < nch - 1)
                def _():
                    out_ref[0, pl.ds(base + k * 384, 384), :] = scr_ref[...]

                @pl.when(k == nch - 1)
                def _():
                    r0 = base + (nch - 1) * 384
                    out_ref[0, r0:base + _HW[s] * 3, :] = scr_ref[:tail_rows, :]


def kernel(x0, x1, x2, idf_logits):
    bs = x0.shape[0]
    xp = []
    for x, hw, hwp in zip((x0, x1, x2), _HW, _HWP):
        xr = x.reshape(bs, 255, hw)
        xp.append(jnp.pad(xr, ((0, 0), (0, 0), (0, hwp - hw))))
    pre = jnp.concatenate(
        [jnp.asarray([-1.0, -1.0, 1.0, 1.0, -1.0], jnp.float32),
         -idf_logits]).reshape(1, 85)
    add_tab = jnp.asarray(_ADD_TAB)

    return pl.pallas_call(
        _body,
        grid=(bs, 61),
        in_specs=[
            pl.BlockSpec((1, 255, 128), lambda b, j: (b, 0, jnp.minimum(j, 45))),
            pl.BlockSpec((1, 255, 128), lambda b, j: (b, 0, jnp.clip(j - 46, 0, 11))),
            pl.BlockSpec((1, 255, 128), lambda b, j: (b, 0, jnp.clip(j - 58, 0, 2))),
            pl.BlockSpec((1, 128, 85), lambda b, j: (j, 0, 0)),
            pl.BlockSpec((1, 85), lambda b, j: (0, 0)),
            pl.BlockSpec((9, 1, 85), lambda b, j: (0, 0, 0)),
        ],
        out_specs=pl.BlockSpec((1, _ROWS, 85), lambda b, j: (b, 0, 0)),
        out_shape=jax.ShapeDtypeStruct((bs, _ROWS, 85), jnp.float32),
        scratch_shapes=[pltpu.VMEM((384, 85), jnp.float32)],
        compiler_params=pltpu.CompilerParams(
            dimension_semantics=("parallel", "arbitrary"),
            vmem_limit_bytes=100 * 2**20),
    )(xp[0], xp[1], xp[2], add_tab, pre, jnp.asarray(_MUL_TAB))


# speedup vs baseline: 2.2914x; 1.3299x over previous
import jax
import jax.numpy as jnp
import numpy as np
from jax.experimental import pallas as pl
from jax.experimental.pallas import tpu as pltpu

_ANCH = (
    ((10.0, 13.0), (16.0, 30.0), (33.0, 23.0)),
    ((30.0, 61.0), (62.0, 45.0), (59.0, 119.0)),
    ((116.0, 90.0), (156.0, 198.0), (373.0, 326.0)),
)
_W = (76, 38, 19)
_HW = (5776, 1444, 361)
_COLS = (256, 256, 384)
_NCH = (23, 6, 1)
_HWP = tuple(c * n for c, n in zip(_COLS, _NCH))
_J0 = (0, 23, 29)
_ROW0 = (0, 17328, 21660)
_STRIDE = (8.0, 16.0, 32.0)
_ROWS = 22743
_TABROWS = 384


def _build_add_table():
    blocks = []
    for s in range(3):
        w, st = _W[s], _STRIDE[s]
        t = np.zeros((_NCH[s], _TABROWS, 85), np.float32)
        hw = np.arange(_COLS[s])
        for k in range(_NCH[s]):
            g = hw + k * _COLS[s]
            t[k, :_COLS[s], 0] = (g % w) * st
            t[k, :_COLS[s], 1] = (g // w) * st
        blocks.append(t)
    return np.concatenate(blocks, 0)


_ADD_TAB = _build_add_table()


def _mul_const(s, a):
    m = np.ones((1, 85), np.float32)
    m[0, 0] = m[0, 1] = _STRIDE[s]
    m[0, 2] = _ANCH[s][a][0]
    m[0, 3] = _ANCH[s][a][1]
    return m


_MUL_TAB = np.concatenate(
    [_mul_const(s, a) for s in range(3) for a in range(3)], 0
).reshape(9, 1, 85)


def _body(x0_ref, x1_ref, x2_ref, add_ref, pre_ref, mul_ref, out_ref, scr_ref):
    j = pl.program_id(1)
    pre = pre_ref[...]

    for s, x_ref in enumerate((x0_ref, x1_ref, x2_ref)):
        jlo, nch, cols = _J0[s], _NCH[s], _COLS[s]

        @pl.when((j >= jlo) & (j < jlo + nch))
        def _(s=s, x_ref=x_ref, jlo=jlo, nch=nch, cols=cols):
            lane = jax.lax.broadcasted_iota(jnp.int32, (cols, 85), 1)
            isexp = (lane == 2) | (lane == 3)
            add = add_ref[0, :cols, :]
            t_all = jnp.transpose(x_ref[0])
            for a in range(3):
                u = t_all[:, a * 85:(a + 1) * 85] * pre
                e = jnp.exp(u)
                nl = jnp.where(isexp, e, 1.0 / (1.0 + e))
                o = nl * mul_ref[s * 3 + a] + add
                scr_ref[a:3 * cols:3, :] = o
            k = j - jlo
            base = _ROW0[s]
            rows = 3 * cols
            tail_rows = (_HW[s] - (nch - 1) * cols) * 3
            if nch == 1:
                out_ref[0, base:base + tail_rows, :] = scr_ref[:tail_rows, :]
            else:
                @pl.when(k < nch - 1)
                def _():
                    out_ref[0, pl.ds(base + k * rows, rows), :] = \
                        scr_ref[:rows, :]

                @pl.when(k == nch - 1)
                def _():
                    r0 = base + (nch - 1) * rows
                    out_ref[0, r0:base + _HW[s] * 3, :] = scr_ref[:tail_rows, :]


def kernel(x0, x1, x2, idf_logits):
    bs = x0.shape[0]
    xp = []
    for x, hw, hwp in zip((x0, x1, x2), _HW, _HWP):
        xr = x.reshape(bs, 255, hw)
        xp.append(jnp.pad(xr, ((0, 0), (0, 0), (0, hwp - hw))))
    pre = jnp.concatenate(
        [jnp.asarray([-1.0, -1.0, 1.0, 1.0, -1.0], jnp.float32),
         -idf_logits]).reshape(1, 85)
    add_tab = jnp.asarray(_ADD_TAB)

    return pl.pallas_call(
        _body,
        grid=(bs, 30),
        in_specs=[
            pl.BlockSpec((1, 255, 256), lambda b, j: (b, 0, jnp.minimum(j, 22))),
            pl.BlockSpec((1, 255, 256), lambda b, j: (b, 0, jnp.clip(j - 23, 0, 5))),
            pl.BlockSpec((1, 255, 384), lambda b, j: (b, 0, 0)),
            pl.BlockSpec((1, _TABROWS, 85), lambda b, j: (j, 0, 0)),
            pl.BlockSpec((1, 85), lambda b, j: (0, 0)),
            pl.BlockSpec((9, 1, 85), lambda b, j: (0, 0, 0)),
        ],
        out_specs=pl.BlockSpec((1, _ROWS, 85), lambda b, j: (b, 0, 0)),
        out_shape=jax.ShapeDtypeStruct((bs, _ROWS, 85), jnp.float32),
        scratch_shapes=[pltpu.VMEM((3 * 384, 85), jnp.float32)],
        compiler_params=pltpu.CompilerParams(
            dimension_semantics=("parallel", "arbitrary"),
            vmem_limit_bytes=100 * 2**20),
    )(xp[0], xp[1], xp[2], add_tab, pre, jnp.asarray(_MUL_TAB))


# speedup vs baseline: 2.9517x; 1.2882x over previous
import jax
import jax.numpy as jnp
import numpy as np
from jax.experimental import pallas as pl
from jax.experimental.pallas import tpu as pltpu

_ANCH = (
    ((10.0, 13.0), (16.0, 30.0), (33.0, 23.0)),
    ((30.0, 61.0), (62.0, 45.0), (59.0, 119.0)),
    ((116.0, 90.0), (156.0, 198.0), (373.0, 326.0)),
)
_W = (76, 38, 19)
_HW = (5776, 1444, 361)
_COLS = (768, 768, 384)
_NCH = (8, 2, 1)
_HWP = tuple(c * n for c, n in zip(_COLS, _NCH))
_J0 = (0, 8, 10)
_ROW0 = (0, 17328, 21660)
_STRIDE = (8.0, 16.0, 32.0)
_ROWS = 22743
_TABROWS = 768


def _build_add_table():
    blocks = []
    for s in range(3):
        w, st = _W[s], _STRIDE[s]
        t = np.zeros((_NCH[s], _TABROWS, 85), np.float32)
        hw = np.arange(_COLS[s])
        for k in range(_NCH[s]):
            g = hw + k * _COLS[s]
            t[k, :_COLS[s], 0] = (g % w) * st
            t[k, :_COLS[s], 1] = (g // w) * st
        blocks.append(t)
    return np.concatenate(blocks, 0)


_ADD_TAB = _build_add_table()


def _mul_const(s, a):
    m = np.ones((1, 85), np.float32)
    m[0, 0] = m[0, 1] = _STRIDE[s]
    m[0, 2] = _ANCH[s][a][0]
    m[0, 3] = _ANCH[s][a][1]
    return m


_MUL_TAB = np.concatenate(
    [_mul_const(s, a) for s in range(3) for a in range(3)], 0
).reshape(9, 1, 85)


def _body(x0_ref, x1_ref, x2_ref, add_ref, pre_ref, mul_ref, out_ref):
    j = pl.program_id(1)
    pre = pre_ref[...]

    for s, x_ref in enumerate((x0_ref, x1_ref, x2_ref)):
        jlo, nch, cols = _J0[s], _NCH[s], _COLS[s]

        @pl.when((j >= jlo) & (j < jlo + nch))
        def _(s=s, x_ref=x_ref, jlo=jlo, nch=nch, cols=cols):
            lane = jax.lax.broadcasted_iota(jnp.int32, (cols, 85), 1)
            isexp = (lane == 2) | (lane == 3)
            add = add_ref[0, :cols, :]
            t_all = jnp.transpose(x_ref[0])
            outs = []
            for a in range(3):
                u = t_all[:, a * 85:(a + 1) * 85] * pre
                e = jnp.exp(u)
                nl = jnp.where(isexp, e, 1.0 / (1.0 + e))
                outs.append(nl * mul_ref[s * 3 + a] + add)
            k = j - jlo
            base = _ROW0[s]
            for kk in range(nch):
                realc = min(_HW[s] - kk * cols, cols)

                @pl.when(k == kk)
                def _(kk=kk, realc=realc):
                    r0 = base + kk * 3 * cols
                    for a in range(3):
                        out_ref[0, r0 + a:r0 + 3 * realc:3, :] = \
                            outs[a][:realc, :]


def kernel(x0, x1, x2, idf_logits):
    bs = x0.shape[0]
    xp = []
    for x, hw, hwp in zip((x0, x1, x2), _HW, _HWP):
        xr = x.reshape(bs, 255, hw)
        xp.append(jnp.pad(xr, ((0, 0), (0, 0), (0, hwp - hw))))
    pre = jnp.concatenate(
        [jnp.asarray([-1.0, -1.0, 1.0, 1.0, -1.0], jnp.float32),
         -idf_logits]).reshape(1, 85)
    add_tab = jnp.asarray(_ADD_TAB)

    return pl.pallas_call(
        _body,
        grid=(bs, 11),
        in_specs=[
            pl.BlockSpec((1, 255, 768), lambda b, j: (b, 0, jnp.minimum(j, 7))),
            pl.BlockSpec((1, 255, 768), lambda b, j: (b, 0, jnp.clip(j - 8, 0, 1))),
            pl.BlockSpec((1, 255, 384), lambda b, j: (b, 0, 0)),
            pl.BlockSpec((1, _TABROWS, 85), lambda b, j: (j, 0, 0)),
            pl.BlockSpec((1, 85), lambda b, j: (0, 0)),
            pl.BlockSpec((9, 1, 85), lambda b, j: (0, 0, 0)),
        ],
        out_specs=pl.BlockSpec((1, _ROWS, 85), lambda b, j: (b, 0, 0)),
        out_shape=jax.ShapeDtypeStruct((bs, _ROWS, 85), jnp.float32),
        compiler_params=pltpu.CompilerParams(
            dimension_semantics=("parallel", "arbitrary"),
            vmem_limit_bytes=100 * 2**20),
    )(xp[0], xp[1], xp[2], add_tab, pre, jnp.asarray(_MUL_TAB))
